# baseline (speedup 1.0000x reference)
import functools

import jax
import jax.numpy as jnp
from jax import lax
from jax.experimental import pallas as pl
from jax.experimental.pallas import tpu as pltpu
from jax.experimental.pallas import tpu_sc as plsc

N = 10000
E = 320000
C = 128
CW = 144
NC = 2
NS = 16
NW = NC * NS
EPW = E // NW
K = 50
CH = EPW // K
RPS = N // NS
BLK = 400


def _make_seg_sum():
    mesh = plsc.VectorSubcoreMesh(core_axis_name="c", subcore_axis_name="s")

    @functools.partial(
        pl.kernel,
        mesh=mesh,
        out_type=jax.ShapeDtypeStruct((NC, N, CW), jnp.float32),
        scratch_types=[
            pltpu.VMEM((CH, K), jnp.int32),
            pltpu.VMEM((CH, K), jnp.int32),
            pltpu.VMEM((2, K, CW), jnp.float32),
            pltpu.VMEM_SHARED((N, CW), jnp.float32),
            pltpu.SemaphoreType.DMA,
            pltpu.SemaphoreType.DMA,
            pltpu.SemaphoreType.DMA,
            pltpu.SemaphoreType.DMA,
        ],
        compiler_params=pltpu.CompilerParams(use_tc_tiling_on_sc=False),
    )
    def seg_sum(table_hbm, sidx_hbm, didx_hbm, zeros_hbm, out_hbm,
                sidx_v, didx_v, rows_v, acc_sh, sem0, sem1, ssem0, ssem1):
        c = lax.axis_index("c")
        s = lax.axis_index("s")
        wid = c * NS + s
        pltpu.sync_copy(zeros_hbm, acc_sh.at[pl.ds(s * RPS, RPS)])
        pltpu.sync_copy(sidx_hbm.at[wid], sidx_v)
        pltpu.sync_copy(didx_hbm.at[wid], didx_v)
        plsc.subcore_barrier()

        pltpu.async_copy(table_hbm.at[sidx_v.at[0]], rows_v.at[0], sem0)
        pltpu.async_copy(table_hbm.at[sidx_v.at[1]], rows_v.at[1], sem1)

        def body(t, carry):
            j = 2 * t
            pltpu.make_async_copy(
                table_hbm.at[sidx_v.at[j]], rows_v.at[0], sem0).wait()
            pltpu.async_copy(rows_v.at[0], acc_sh.at[didx_v.at[j]], ssem0,
                             add=True)
            pltpu.make_async_copy(
                table_hbm.at[sidx_v.at[j + 1]], rows_v.at[1], sem1).wait()
            pltpu.async_copy(rows_v.at[1], acc_sh.at[didx_v.at[j + 1]], ssem1,
                             add=True)
            pltpu.make_async_copy(
                rows_v.at[0], acc_sh.at[didx_v.at[j]], ssem0).wait()

            @pl.when(j + 2 < CH)
            def _():
                pltpu.async_copy(
                    table_hbm.at[sidx_v.at[j + 2]], rows_v.at[0], sem0)

            pltpu.make_async_copy(
                rows_v.at[1], acc_sh.at[didx_v.at[j + 1]], ssem1).wait()

            @pl.when(j + 3 < CH)
            def _():
                pltpu.async_copy(
                    table_hbm.at[sidx_v.at[j + 3]], rows_v.at[1], sem1)
            return carry

        lax.fori_loop(0, CH // 2, body, 0)

        plsc.subcore_barrier()
        pltpu.sync_copy(acc_sh.at[pl.ds(s * RPS, RPS)],
                        out_hbm.at[c, pl.ds(s * RPS, RPS)])

    return seg_sum


_seg_sum = _make_seg_sum()


def _merge_agg(p0, p1):
    acc = p0 + p1
    cnt = jnp.maximum(acc[:, C:C + 1], 1.0)
    return acc[:, :C] * (1.0 / cnt)


def _layer_norm(r, g, be):
    mu = jnp.mean(r, axis=1, keepdims=True)
    d = r - mu
    var = jnp.mean(d * d, axis=1, keepdims=True)
    return d * lax.rsqrt(var + 1e-5) * g + be


def _silu(x):
    return x * (1.0 / (1.0 + jnp.exp(-x)))


def _cls_body(h_ref, p0_ref, p1_ref, w1a_ref, w1b_ref, b1_ref, w2_ref, b2_ref,
              g_ref, be_ref, out_ref, aug_ref):
    hp = h_ref[...]
    agg = _merge_agg(p0_ref[0], p1_ref[0])
    u1 = _silu(hp @ w1a_ref[...] + agg @ w1b_ref[...] + b1_ref[...])
    u = u1 @ w2_ref[...] + b2_ref[...]
    y = _layer_norm(hp + u, g_ref[...], be_ref[...])
    out_ref[...] = y
    aug_ref[:, :C] = y
    lane = lax.broadcasted_iota(jnp.int32, (BLK, CW - C), 1)
    aug_ref[:, C:] = jnp.where(lane == 0, 1.0, 0.0)


def _lit_body(h_ref, p0_ref, p1_ref, w1a_ref, w1b_ref, w1c_ref, b1_ref,
              w2_ref, b2_ref, g_ref, be_ref, out_ref):
    hp = h_ref[...]
    agg = _merge_agg(p0_ref[0], p1_ref[0])
    up = pltpu.roll(hp, BLK - 1, 0)
    dn = pltpu.roll(hp, 1, 0)
    parity = lax.broadcasted_iota(jnp.int32, (BLK, C), 0) % 2
    fl = jnp.where(parity == 0, up, dn)
    u1 = _silu(hp @ w1a_ref[...] + fl @ w1b_ref[...] + agg @ w1c_ref[...]
               + b1_ref[...])
    v = u1 @ w2_ref[...] + b2_ref[...]
    out_ref[...] = _layer_norm(hp + v, g_ref[...], be_ref[...])


def _row_spec(w):
    return pl.BlockSpec((BLK, w), lambda i: (i, 0))


def _acc_spec(slab):
    return pl.BlockSpec((1, BLK, CW), lambda i, s=slab: (s, i, 0))


def _full_spec(a):
    return pl.BlockSpec(a.shape, lambda i: (0,) * a.ndim)


def _mlp_cls(h, acc, w1a, w1b, b1, w2, b2, g, be):
    return pl.pallas_call(
        _cls_body,
        grid=(N // BLK,),
        in_specs=[_row_spec(C), _acc_spec(0), _acc_spec(1),
                  _full_spec(w1a), _full_spec(w1b), _full_spec(b1),
                  _full_spec(w2), _full_spec(b2), _full_spec(g),
                  _full_spec(be)],
        out_specs=[_row_spec(C), _row_spec(CW)],
        out_shape=[jax.ShapeDtypeStruct((N, C), jnp.float32),
                   jax.ShapeDtypeStruct((N, CW), jnp.float32)],
    )(h, acc, acc, w1a, w1b, b1, w2, b2, g, be)


def _mlp_lit(h, acc, w1a, w1b, w1c, b1, w2, b2, g, be):
    return pl.pallas_call(
        _lit_body,
        grid=(N // BLK,),
        in_specs=[_row_spec(C), _acc_spec(0), _acc_spec(1),
                  _full_spec(w1a), _full_spec(w1b), _full_spec(w1c),
                  _full_spec(b1), _full_spec(w2), _full_spec(b2),
                  _full_spec(g), _full_spec(be)],
        out_specs=_row_spec(C),
        out_shape=jax.ShapeDtypeStruct((N, C), jnp.float32),
    )(h, acc, acc, w1a, w1b, w1c, b1, w2, b2, g, be)


@jax.jit
def kernel(h_lit, h_cls, edge_index, W1_cls, b1_cls, W2_cls, b2_cls,
           W1_lit, b1_lit, W2_lit, b2_lit, g_cls, be_cls, g_lit, be_lit):
    ei = edge_index.astype(jnp.int32)
    cls_idx = ei[0].reshape(NW, CH, K)
    lit_idx = ei[1].reshape(NW, CH, K)
    zeros = jnp.zeros((RPS, CW), jnp.float32)
    pad = jnp.tile(
        jnp.concatenate([jnp.ones((1, 1), jnp.float32),
                         jnp.zeros((1, CW - C - 1), jnp.float32)], axis=1),
        (N, 1))
    h_lit_aug = jnp.concatenate([h_lit, pad], axis=1)

    b1c = b1_cls.reshape(1, C)
    b2c = b2_cls.reshape(1, C)
    b1l = b1_lit.reshape(1, C)
    b2l = b2_lit.reshape(1, C)
    gc = g_cls.reshape(1, C)
    bec = be_cls.reshape(1, C)
    gl = g_lit.reshape(1, C)
    bel = be_lit.reshape(1, C)

    acc_a = _seg_sum(h_lit_aug, lit_idx, cls_idx, zeros)
    h_cls_new, h_cls_aug = _mlp_cls(
        h_cls, acc_a, W1_cls[:C], W1_cls[C:], b1c, W2_cls, b2c, gc, bec)

    acc_b = _seg_sum(h_cls_aug, cls_idx, lit_idx, zeros)
    h_lit_new = _mlp_lit(
        h_lit, acc_b, W1_lit[:C], W1_lit[C:2 * C], W1_lit[2 * C:], b1l,
        W2_lit, b2l, gl, bel)

    return (h_lit_new, h_cls_new)


# speedup vs baseline: 1.4912x; 1.4912x over previous
import functools

import jax
import jax.numpy as jnp
from jax import lax
from jax.experimental import pallas as pl
from jax.experimental.pallas import tpu as pltpu
from jax.experimental.pallas import tpu_sc as plsc

N = 10000
E = 320000
C = 128
CW = 144
NC = 2
NS = 16
NW = NC * NS
EPW = E // NW
K = 125
CH = EPW // K
RPS = N // NS
BLK = 400


def _make_seg_sum():
    mesh = plsc.VectorSubcoreMesh(core_axis_name="c", subcore_axis_name="s")

    @functools.partial(
        pl.kernel,
        mesh=mesh,
        out_type=jax.ShapeDtypeStruct((NC, N, CW), jnp.bfloat16),
        scratch_types=[
            pltpu.VMEM((CH, K), jnp.int32),
            pltpu.VMEM((CH, K), jnp.int32),
            pltpu.VMEM((2, K, CW), jnp.bfloat16),
            pltpu.VMEM_SHARED((N, CW), jnp.bfloat16),
            pltpu.SemaphoreType.DMA,
            pltpu.SemaphoreType.DMA,
        ],
        compiler_params=pltpu.CompilerParams(use_tc_tiling_on_sc=False),
    )
    def seg_sum(table_hbm, sidx_hbm, didx_hbm, zeros_hbm, out_hbm,
                sidx_v, didx_v, rows_v, acc_sh, sem0, sem1):
        c = lax.axis_index("c")
        s = lax.axis_index("s")
        wid = c * NS + s
        pltpu.sync_copy(zeros_hbm, acc_sh.at[pl.ds(s * RPS, RPS)])
        pltpu.sync_copy(sidx_hbm.at[wid], sidx_v)
        pltpu.sync_copy(didx_hbm.at[wid], didx_v)
        plsc.subcore_barrier()

        pltpu.async_copy(table_hbm.at[sidx_v.at[0]], rows_v.at[0], sem0)

        def body(t, carry):
            j = 2 * t
            pltpu.async_copy(table_hbm.at[sidx_v.at[j + 1]], rows_v.at[1], sem1)
            pltpu.make_async_copy(
                table_hbm.at[sidx_v.at[j]], rows_v.at[0], sem0).wait()
            pltpu.sync_copy(rows_v.at[0], acc_sh.at[didx_v.at[j]], add=True)

            @pl.when(j + 2 < CH)
            def _():
                pltpu.async_copy(
                    table_hbm.at[sidx_v.at[j + 2]], rows_v.at[0], sem0)

            pltpu.make_async_copy(
                table_hbm.at[sidx_v.at[j + 1]], rows_v.at[1], sem1).wait()
            pltpu.sync_copy(rows_v.at[1], acc_sh.at[didx_v.at[j + 1]], add=True)
            return carry

        lax.fori_loop(0, CH // 2, body, 0)

        plsc.subcore_barrier()
        pltpu.sync_copy(acc_sh.at[pl.ds(s * RPS, RPS)],
                        out_hbm.at[c, pl.ds(s * RPS, RPS)])

    return seg_sum


_seg_sum = _make_seg_sum()


def _merge_agg(p0, p1):
    acc = p0.astype(jnp.float32) + p1.astype(jnp.float32)
    cnt = jnp.maximum(acc[:, C:C + 1], 1.0)
    return acc[:, :C] * (1.0 / cnt)


def _layer_norm(r, g, be):
    mu = jnp.mean(r, axis=1, keepdims=True)
    d = r - mu
    var = jnp.mean(d * d, axis=1, keepdims=True)
    return d * lax.rsqrt(var + 1e-5) * g + be


def _silu(x):
    return x * (1.0 / (1.0 + jnp.exp(-x)))


def _cls_body(h_ref, p0_ref, p1_ref, w1a_ref, w1b_ref, b1_ref, w2_ref, b2_ref,
              g_ref, be_ref, out_ref, aug_ref):
    hp = h_ref[...]
    agg = _merge_agg(p0_ref[0], p1_ref[0])
    u1 = _silu(hp @ w1a_ref[...] + agg @ w1b_ref[...] + b1_ref[...])
    u = u1 @ w2_ref[...] + b2_ref[...]
    y = _layer_norm(hp + u, g_ref[...], be_ref[...])
    out_ref[...] = y
    aug_ref[:, :C] = y.astype(jnp.bfloat16)
    lane = lax.broadcasted_iota(jnp.int32, (BLK, CW - C), 1)
    aug_ref[:, C:] = jnp.where(lane == 0, 1.0, 0.0).astype(jnp.bfloat16)


def _lit_body(h_ref, p0_ref, p1_ref, w1a_ref, w1b_ref, w1c_ref, b1_ref,
              w2_ref, b2_ref, g_ref, be_ref, out_ref):
    hp = h_ref[...]
    agg = _merge_agg(p0_ref[0], p1_ref[0])
    up = pltpu.roll(hp, BLK - 1, 0)
    dn = pltpu.roll(hp, 1, 0)
    parity = lax.broadcasted_iota(jnp.int32, (BLK, C), 0) % 2
    fl = jnp.where(parity == 0, up, dn)
    u1 = _silu(hp @ w1a_ref[...] + fl @ w1b_ref[...] + agg @ w1c_ref[...]
               + b1_ref[...])
    v = u1 @ w2_ref[...] + b2_ref[...]
    out_ref[...] = _layer_norm(hp + v, g_ref[...], be_ref[...])


def _row_spec(w):
    return pl.BlockSpec((BLK, w), lambda i: (i, 0))


def _acc_spec(slab):
    return pl.BlockSpec((1, BLK, CW), lambda i, s=slab: (s, i, 0))


def _full_spec(a):
    return pl.BlockSpec(a.shape, lambda i: (0,) * a.ndim)


def _mlp_cls(h, acc, w1a, w1b, b1, w2, b2, g, be):
    return pl.pallas_call(
        _cls_body,
        grid=(N // BLK,),
        in_specs=[_row_spec(C), _acc_spec(0), _acc_spec(1),
                  _full_spec(w1a), _full_spec(w1b), _full_spec(b1),
                  _full_spec(w2), _full_spec(b2), _full_spec(g),
                  _full_spec(be)],
        out_specs=[_row_spec(C), _row_spec(CW)],
        out_shape=[jax.ShapeDtypeStruct((N, C), jnp.float32),
                   jax.ShapeDtypeStruct((N, CW), jnp.bfloat16)],
    )(h, acc, acc, w1a, w1b, b1, w2, b2, g, be)


def _mlp_lit(h, acc, w1a, w1b, w1c, b1, w2, b2, g, be):
    return pl.pallas_call(
        _lit_body,
        grid=(N // BLK,),
        in_specs=[_row_spec(C), _acc_spec(0), _acc_spec(1),
                  _full_spec(w1a), _full_spec(w1b), _full_spec(w1c),
                  _full_spec(b1), _full_spec(w2), _full_spec(b2),
                  _full_spec(g), _full_spec(be)],
        out_specs=_row_spec(C),
        out_shape=jax.ShapeDtypeStruct((N, C), jnp.float32),
    )(h, acc, acc, w1a, w1b, w1c, b1, w2, b2, g, be)


@jax.jit
def kernel(h_lit, h_cls, edge_index, W1_cls, b1_cls, W2_cls, b2_cls,
           W1_lit, b1_lit, W2_lit, b2_lit, g_cls, be_cls, g_lit, be_lit):
    ei = edge_index.astype(jnp.int32)
    cls_idx = ei[0].reshape(NW, CH, K)
    lit_idx = ei[1].reshape(NW, CH, K)
    zeros = jnp.zeros((RPS, CW), jnp.bfloat16)
    pad = jnp.tile(
        jnp.concatenate([jnp.ones((1, 1), jnp.float32),
                         jnp.zeros((1, CW - C - 1), jnp.float32)], axis=1),
        (N, 1))
    h_lit_aug = jnp.concatenate([h_lit, pad], axis=1).astype(jnp.bfloat16)

    b1c = b1_cls.reshape(1, C)
    b2c = b2_cls.reshape(1, C)
    b1l = b1_lit.reshape(1, C)
    b2l = b2_lit.reshape(1, C)
    gc = g_cls.reshape(1, C)
    bec = be_cls.reshape(1, C)
    gl = g_lit.reshape(1, C)
    bel = be_lit.reshape(1, C)

    acc_a = _seg_sum(h_lit_aug, lit_idx, cls_idx, zeros)
    h_cls_new, h_cls_aug = _mlp_cls(
        h_cls, acc_a, W1_cls[:C], W1_cls[C:], b1c, W2_cls, b2c, gc, bec)

    acc_b = _seg_sum(h_cls_aug, cls_idx, lit_idx, zeros)
    h_lit_new = _mlp_lit(
        h_lit, acc_b, W1_lit[:C], W1_lit[C:2 * C], W1_lit[2 * C:], b1l,
        W2_lit, b2l, gl, bel)

    return (h_lit_new, h_cls_new)


# speedup vs baseline: 1.6340x; 1.0958x over previous
import functools

import jax
import jax.numpy as jnp
from jax import lax
from jax.experimental import pallas as pl
from jax.experimental.pallas import tpu as pltpu
from jax.experimental.pallas import tpu_sc as plsc

N = 10000
E = 320000
C = 128
CC = 32
NC = 2
NS = 16
NW = NC * NS
EPW = E // NW
K = 125
CH = EPW // K
RPS = N // NS
BLK = 400


def _make_seg_sum():
    mesh = plsc.VectorSubcoreMesh(core_axis_name="c", subcore_axis_name="s")

    @functools.partial(
        pl.kernel,
        mesh=mesh,
        out_type=[jax.ShapeDtypeStruct((NC, N, C), jnp.bfloat16),
                  jax.ShapeDtypeStruct((NC, N, CC), jnp.bfloat16)],
        scratch_types=[
            pltpu.VMEM((CH, K), jnp.int32),
            pltpu.VMEM((CH, K), jnp.int32),
            pltpu.VMEM((2, K, C), jnp.bfloat16),
            pltpu.VMEM((K, CC), jnp.bfloat16),
            pltpu.VMEM_SHARED((N, C), jnp.bfloat16),
            pltpu.VMEM_SHARED((N, CC), jnp.bfloat16),
            pltpu.SemaphoreType.DMA,
            pltpu.SemaphoreType.DMA,
        ],
        compiler_params=pltpu.CompilerParams(use_tc_tiling_on_sc=False),
    )
    def seg_sum(table_hbm, sidx_hbm, didx_hbm, zf_hbm, zc_hbm, ones_hbm,
                feat_out, cnt_out, sidx_v, didx_v, rows_v, ones_v,
                accf_sh, accc_sh, sem0, sem1):
        c = lax.axis_index("c")
        s = lax.axis_index("s")
        wid = c * NS + s
        pltpu.sync_copy(zf_hbm, accf_sh.at[pl.ds(s * RPS, RPS)])
        pltpu.sync_copy(zc_hbm, accc_sh.at[pl.ds(s * RPS, RPS)])
        pltpu.sync_copy(ones_hbm, ones_v)
        pltpu.sync_copy(sidx_hbm.at[wid], sidx_v)
        pltpu.sync_copy(didx_hbm.at[wid], didx_v)
        plsc.subcore_barrier()

        pltpu.async_copy(table_hbm.at[sidx_v.at[0]], rows_v.at[0], sem0)

        def body(t, carry):
            j = 2 * t
            pltpu.async_copy(table_hbm.at[sidx_v.at[j + 1]], rows_v.at[1], sem1)
            pltpu.make_async_copy(
                table_hbm.at[sidx_v.at[j]], rows_v.at[0], sem0).wait()
            pltpu.sync_copy(rows_v.at[0], accf_sh.at[didx_v.at[j]], add=True)
            pltpu.sync_copy(ones_v, accc_sh.at[didx_v.at[j]], add=True)

            @pl.when(j + 2 < CH)
            def _():
                pltpu.async_copy(
                    table_hbm.at[sidx_v.at[j + 2]], rows_v.at[0], sem0)

            pltpu.make_async_copy(
                table_hbm.at[sidx_v.at[j + 1]], rows_v.at[1], sem1).wait()
            pltpu.sync_copy(rows_v.at[1], accf_sh.at[didx_v.at[j + 1]],
                            add=True)
            pltpu.sync_copy(ones_v, accc_sh.at[didx_v.at[j + 1]], add=True)
            return carry

        lax.fori_loop(0, CH // 2, body, 0)

        plsc.subcore_barrier()
        pltpu.sync_copy(accf_sh.at[pl.ds(s * RPS, RPS)],
                        feat_out.at[c, pl.ds(s * RPS, RPS)])
        pltpu.sync_copy(accc_sh.at[pl.ds(s * RPS, RPS)],
                        cnt_out.at[c, pl.ds(s * RPS, RPS)])

    return seg_sum


_seg_sum = _make_seg_sum()


def _merge_agg(f0, f1, c0, c1):
    f = f0.astype(jnp.float32) + f1.astype(jnp.float32)
    cnt = c0[:, :1].astype(jnp.float32) + c1[:, :1].astype(jnp.float32)
    return f * (1.0 / jnp.maximum(cnt, 1.0))


def _layer_norm(r, g, be):
    mu = jnp.mean(r, axis=1, keepdims=True)
    d = r - mu
    var = jnp.mean(d * d, axis=1, keepdims=True)
    return d * lax.rsqrt(var + 1e-5) * g + be


def _silu(x):
    return x * (1.0 / (1.0 + jnp.exp(-x)))


def _cls_body(h_ref, f0_ref, f1_ref, c0_ref, c1_ref, w1a_ref, w1b_ref, b1_ref,
              w2_ref, b2_ref, g_ref, be_ref, out_ref, tab_ref):
    hp = h_ref[...]
    agg = _merge_agg(f0_ref[0], f1_ref[0], c0_ref[0], c1_ref[0])
    u1 = _silu(hp @ w1a_ref[...] + agg @ w1b_ref[...] + b1_ref[...])
    u = u1 @ w2_ref[...] + b2_ref[...]
    y = _layer_norm(hp + u, g_ref[...], be_ref[...])
    out_ref[...] = y
    tab_ref[...] = y.astype(jnp.bfloat16)


def _lit_body(h_ref, f0_ref, f1_ref, c0_ref, c1_ref, w1a_ref, w1b_ref,
              w1c_ref, b1_ref, w2_ref, b2_ref, g_ref, be_ref, out_ref):
    hp = h_ref[...]
    agg = _merge_agg(f0_ref[0], f1_ref[0], c0_ref[0], c1_ref[0])
    up = pltpu.roll(hp, BLK - 1, 0)
    dn = pltpu.roll(hp, 1, 0)
    parity = lax.broadcasted_iota(jnp.int32, (BLK, C), 0) % 2
    fl = jnp.where(parity == 0, up, dn)
    u1 = _silu(hp @ w1a_ref[...] + fl @ w1b_ref[...] + agg @ w1c_ref[...]
               + b1_ref[...])
    v = u1 @ w2_ref[...] + b2_ref[...]
    out_ref[...] = _layer_norm(hp + v, g_ref[...], be_ref[...])


def _row_spec(w):
    return pl.BlockSpec((BLK, w), lambda i: (i, 0))


def _acc_spec(slab, w):
    return pl.BlockSpec((1, BLK, w), lambda i, s=slab: (s, i, 0))


def _full_spec(a):
    return pl.BlockSpec(a.shape, lambda i: (0,) * a.ndim)


def _mlp_cls(h, feat, cnt, w1a, w1b, b1, w2, b2, g, be):
    return pl.pallas_call(
        _cls_body,
        grid=(N // BLK,),
        in_specs=[_row_spec(C), _acc_spec(0, C), _acc_spec(1, C),
                  _acc_spec(0, CC), _acc_spec(1, CC),
                  _full_spec(w1a), _full_spec(w1b), _full_spec(b1),
                  _full_spec(w2), _full_spec(b2), _full_spec(g),
                  _full_spec(be)],
        out_specs=[_row_spec(C), _row_spec(C)],
        out_shape=[jax.ShapeDtypeStruct((N, C), jnp.float32),
                   jax.ShapeDtypeStruct((N, C), jnp.bfloat16)],
    )(h, feat, feat, cnt, cnt, w1a, w1b, b1, w2, b2, g, be)


def _mlp_lit(h, feat, cnt, w1a, w1b, w1c, b1, w2, b2, g, be):
    return pl.pallas_call(
        _lit_body,
        grid=(N // BLK,),
        in_specs=[_row_spec(C), _acc_spec(0, C), _acc_spec(1, C),
                  _acc_spec(0, CC), _acc_spec(1, CC),
                  _full_spec(w1a), _full_spec(w1b), _full_spec(w1c),
                  _full_spec(b1), _full_spec(w2), _full_spec(b2),
                  _full_spec(g), _full_spec(be)],
        out_specs=_row_spec(C),
        out_shape=jax.ShapeDtypeStruct((N, C), jnp.float32),
    )(h, feat, feat, cnt, cnt, w1a, w1b, w1c, b1, w2, b2, g, be)


@jax.jit
def kernel(h_lit, h_cls, edge_index, W1_cls, b1_cls, W2_cls, b2_cls,
           W1_lit, b1_lit, W2_lit, b2_lit, g_cls, be_cls, g_lit, be_lit):
    ei = edge_index.astype(jnp.int32).reshape(2, NW, CH, K)
    cls_idx = ei[0]
    lit_idx = ei[1]
    zf = jnp.zeros((RPS, C), jnp.bfloat16)
    zc = jnp.zeros((RPS, CC), jnp.bfloat16)
    ones = jnp.ones((K, CC), jnp.bfloat16)
    h_lit_b = h_lit.astype(jnp.bfloat16)

    b1c = b1_cls.reshape(1, C)
    b2c = b2_cls.reshape(1, C)
    b1l = b1_lit.reshape(1, C)
    b2l = b2_lit.reshape(1, C)
    gc = g_cls.reshape(1, C)
    bec = be_cls.reshape(1, C)
    gl = g_lit.reshape(1, C)
    bel = be_lit.reshape(1, C)

    feat_a, cnt_a = _seg_sum(h_lit_b, lit_idx, cls_idx, zf, zc, ones)
    h_cls_new, h_cls_b = _mlp_cls(
        h_cls, feat_a, cnt_a, W1_cls[:C], W1_cls[C:], b1c, W2_cls, b2c,
        gc, bec)

    feat_b, cnt_b = _seg_sum(h_cls_b, cls_idx, lit_idx, zf, zc, ones)
    h_lit_new = _mlp_lit(
        h_lit, feat_b, cnt_b, W1_lit[:C], W1_lit[C:2 * C], W1_lit[2 * C:],
        b1l, W2_lit, b2l, gl, bel)

    return (h_lit_new, h_cls_new)
